# baseline (speedup 1.0000x reference)
import jax
import jax.numpy as jnp
from jax.experimental import pallas as pl

N = 10000
H = 8
C = 64


def _gat(x, edge_index, W, att_src, att_dst, bias, heads, out_ch, concat):
    n = x.shape[0]
    loop = jnp.arange(n, dtype=edge_index.dtype)
    src = jnp.concatenate([edge_index[0], loop])
    dst = jnp.concatenate([edge_index[1], loop])
    h = (x @ W).reshape(n, heads, out_ch)
    a_src = (h * att_src[None, :, :]).sum(-1)
    a_dst = (h * att_dst[None, :, :]).sum(-1)
    e = a_src[src] + a_dst[dst]
    e = jax.nn.leaky_relu(e, 0.2)
    m = jax.ops.segment_max(e, dst, num_segments=n)
    m = jnp.where(jnp.isfinite(m), m, 0.0)
    alpha = jnp.exp(e - m[dst])
    denom = jax.ops.segment_sum(alpha, dst, num_segments=n)
    alpha = alpha / (denom[dst] + 1e-16)
    msg = h[src] * alpha[:, :, None]
    out = jax.ops.segment_sum(msg, dst, num_segments=n)
    if concat:
        out = out.reshape(n, heads * out_ch)
    else:
        out = out.mean(axis=1)
    return out + bias


def _copy_body(x_ref, o_ref):
    o_ref[...] = x_ref[...]


def kernel(x, edge_index, W1, as1, ad1, b1, W2, as2, ad2, b2, W3, as3, ad3, b3, linW, linb):
    h = jax.nn.elu(_gat(x, edge_index, W1, as1, ad1, b1, H, C, True))
    h = jax.nn.elu(_gat(h, edge_index, W2, as2, ad2, b2, H, C, True))
    h = _gat(h, edge_index, W3, as3, ad3, b3, 1, C, False)
    logits = h @ linW + linb
    out = jax.nn.log_softmax(logits, axis=1)
    out = pl.pallas_call(
        _copy_body,
        out_shape=jax.ShapeDtypeStruct(out.shape, out.dtype),
    )(out)
    return out


# speedup vs baseline: 13.8788x; 13.8788x over previous
import functools

import jax
import jax.numpy as jnp
from jax import lax
from jax.experimental import pallas as pl
from jax.experimental.pallas import tpu as pltpu
from jax.experimental.pallas import tpu_sc as plsc

N = 10000
E = 320000
F_IN = 128
H = 8
C = 64

NP = 10240
NSC = 2
NTILES = 16
EP = 327680
TILE_E = EP // NSC // NTILES
SUB = 2048
KCH = 128
BM = 512
ROWS_PT = NP // NTILES


def _lrelu(v):
    return jnp.where(v >= 0.0, v, 0.2 * v)


def _mm_body(d0, z_ref, w_ref, o_ref, am_ref):
    o = jnp.dot(z_ref[...], w_ref[...], preferred_element_type=jnp.float32)
    o_ref[...] = o
    bm = jnp.max(o[:, d0:d0 + 128], axis=0)
    bm8 = jnp.broadcast_to(bm[None, :], (8, 128))
    @pl.when(pl.program_id(0) == 0)
    def _():
        am_ref[...] = bm8
    @pl.when(pl.program_id(0) > 0)
    def _():
        am_ref[...] = jnp.maximum(am_ref[...], bm8)


def _mm(z, w_aug, d0):
    k = z.shape[1]
    dw = d0 + 128
    return pl.pallas_call(
        functools.partial(_mm_body, d0),
        grid=(NP // BM,),
        in_specs=[
            pl.BlockSpec((BM, k), lambda i: (i, 0)),
            pl.BlockSpec((k, dw), lambda i: (0, 0)),
        ],
        out_specs=[
            pl.BlockSpec((BM, dw), lambda i: (i, 0)),
            pl.BlockSpec((8, 128), lambda i: (0, 0)),
        ],
        out_shape=[
            jax.ShapeDtypeStruct((NP, dw), jnp.float32),
            jax.ShapeDtypeStruct((8, 128), jnp.float32),
        ],
    )(z, w_aug)


def _fin_mm_body(d0p, hp, d0n, op_ref, acc_ref, den_ref, am_ref, b_ref,
                 w_ref, o_ref, amo_ref):
    op = op_ref[...]
    h = op[:, :d0p]
    a_s = op[:, d0p:d0p + hp]
    a_d = op[:, d0p + 8:d0p + 8 + hp]
    a_glob = am_ref[0:1, 0:hp]
    w_self = jnp.exp(_lrelu(a_s + a_d) - _lrelu(a_glob + a_d))
    dent = den_ref[:, 0:hp] + w_self
    cph = d0p // hp
    sel = (lax.broadcasted_iota(jnp.int32, (hp, d0p), 1) // cph
           == lax.broadcasted_iota(jnp.int32, (hp, d0p), 0)
           ).astype(jnp.float32)
    wf = jnp.dot(w_self, sel, preferred_element_type=jnp.float32)
    df = jnp.dot(dent, sel, preferred_element_type=jnp.float32)
    z = (acc_ref[...] + wf * h) / df + b_ref[0:1, :]
    z = jnp.where(z > 0.0, z, jnp.exp(z) - 1.0)
    o = jnp.dot(z, w_ref[...], preferred_element_type=jnp.float32)
    o_ref[...] = o
    bm = jnp.max(o[:, d0n:d0n + 128], axis=0)
    bm8 = jnp.broadcast_to(bm[None, :], (8, 128))
    @pl.when(pl.program_id(0) == 0)
    def _():
        amo_ref[...] = bm8
    @pl.when(pl.program_id(0) > 0)
    def _():
        amo_ref[...] = jnp.maximum(amo_ref[...], bm8)


def _fin_mm(o_prev, acc, den, amax, bias2d, w_aug, d0p, hp):
    dwp = o_prev.shape[1]
    d0n = w_aug.shape[1] - 128
    return pl.pallas_call(
        functools.partial(_fin_mm_body, d0p, hp, d0n),
        grid=(NP // BM,),
        in_specs=[
            pl.BlockSpec((BM, dwp), lambda i: (i, 0)),
            pl.BlockSpec((BM, d0p), lambda i: (i, 0)),
            pl.BlockSpec((BM, 16), lambda i: (i, 0)),
            pl.BlockSpec((8, 128), lambda i: (0, 0)),
            pl.BlockSpec((1, d0p), lambda i: (0, 0)),
            pl.BlockSpec((w_aug.shape[0], w_aug.shape[1]), lambda i: (0, 0)),
        ],
        out_specs=[
            pl.BlockSpec((BM, w_aug.shape[1]), lambda i: (i, 0)),
            pl.BlockSpec((8, 128), lambda i: (0, 0)),
        ],
        out_shape=[
            jax.ShapeDtypeStruct((NP, w_aug.shape[1]), jnp.float32),
            jax.ShapeDtypeStruct((8, 128), jnp.float32),
        ],
    )(o_prev, acc, den, amax, bias2d, w_aug)


def _head_body(o3_ref, acc_ref, den_ref, am_ref, b_ref, lw_ref, lb_ref,
               out_ref):
    o3 = o3_ref[...]
    h = o3[:, :C]
    a_s = o3[:, C:C + 1]
    a_d = o3[:, C + 8:C + 9]
    a_glob = am_ref[0:1, 0:1]
    w_self = jnp.exp(_lrelu(a_s + a_d) - _lrelu(a_glob + a_d))
    dent = den_ref[:, 0:1] + w_self
    z = (acc_ref[...] + w_self * h) / dent + b_ref[0:1, :]
    lg = jnp.dot(z, lw_ref[...], preferred_element_type=jnp.float32)
    lg = lg + lb_ref[0:1, :]
    mx = jnp.max(lg, axis=1, keepdims=True)
    s = lg - mx
    out_ref[...] = s - jnp.log(jnp.sum(jnp.exp(s), axis=1, keepdims=True))


def _head(o3, acc3, den3, am3, b3_2d, lin_w, lin_b2d):
    dwp = o3.shape[1]
    return pl.pallas_call(
        _head_body,
        grid=(NP // BM,),
        in_specs=[
            pl.BlockSpec((BM, dwp), lambda i: (i, 0)),
            pl.BlockSpec((BM, C), lambda i: (i, 0)),
            pl.BlockSpec((BM, 16), lambda i: (i, 0)),
            pl.BlockSpec((8, 128), lambda i: (0, 0)),
            pl.BlockSpec((1, C), lambda i: (0, 0)),
            pl.BlockSpec((C, 2), lambda i: (0, 0)),
            pl.BlockSpec((1, 2), lambda i: (0, 0)),
        ],
        out_specs=pl.BlockSpec((BM, 2), lambda i: (i, 0)),
        out_shape=jax.ShapeDtypeStruct((NP, 2), jnp.float32),
    )(o3, acc3, den3, am3, b3_2d, lin_w, lin_b2d)


def _sc_body(nslab, slabw, hh, *refs):
    hslabs = list(refs[:nslab])
    (asd_hbm, amax_hbm, src_hbm, dst_hbm, acc_out, den_out,
     stage_s, stage_d, gidx_s, gidx_d, sidx,
     rows, arow, brow, wbuf, amv, acc_sh, den_sh, sem) = refs[nslab:]
    hps = slabw // C
    cid = lax.axis_index("c")
    sid = lax.axis_index("s")
    pltpu.sync_copy(amax_hbm, amv)
    amv_val = amv[...]
    ebase = (cid * NTILES + sid) * TILE_E

    for p in range(nslab):
        def _z_rows(j, c):
            for g in range(slabw // 16):
                rows[j, pl.ds(g * 16, 16)] = jnp.zeros((16,), jnp.float32)
            if p == 0:
                wbuf[j, :] = jnp.zeros((16,), jnp.float32)
            return c
        lax.fori_loop(0, KCH, _z_rows, 0)
        base = sid * ROWS_PT
        for r in range(ROWS_PT // KCH):
            pltpu.sync_copy(rows.at[pl.ds(0, KCH)],
                            acc_sh.at[pl.ds(base + r * KCH, KCH)])
            if p == 0:
                pltpu.sync_copy(wbuf.at[pl.ds(0, KCH)],
                                den_sh.at[pl.ds(base + r * KCH, KCH)])
        plsc.subcore_barrier()

        def _sub(si, c):
            off = pl.multiple_of(ebase + si * SUB, 8)
            pltpu.sync_copy(src_hbm.at[pl.ds(off, SUB)], stage_s)
            pltpu.sync_copy(dst_hbm.at[pl.ds(off, SUB)], stage_d)

            def _chunk(ci, c):
                c0 = ci * KCH
                for g in range(KCH // 16):
                    sv = stage_s[pl.ds(c0 + g * 16, 16)]
                    dv = stage_d[pl.ds(c0 + g * 16, 16)]
                    gidx_s[0, pl.ds(g * 16, 16)] = sv
                    gidx_d[0, pl.ds(g * 16, 16)] = dv
                    sidx[0, pl.ds(g * 16, 16)] = dv
                d1 = pltpu.async_copy(hslabs[p].at[gidx_s.at[0]], rows, sem)
                d2 = pltpu.async_copy(asd_hbm.at[gidx_s.at[0]], arow, sem)
                d3 = pltpu.async_copy(asd_hbm.at[gidx_d.at[0]], brow, sem)
                d1.wait()
                d2.wait()
                d3.wait()

                def _edge(j, _):
                    lane_i = lax.iota(jnp.int32, 16)
                    lane_ok = jnp.where(lane_i < hh, 1.0, 0.0)
                    av = arow[j, :]
                    bv = brow[j, :]
                    adv = jnp.take(bv, (lane_i + 8) % 16)
                    e = av + adv
                    e = jnp.where(e >= 0., e, 0.2 * e)
                    t = amv_val + adv
                    t = jnp.where(t >= 0., t, 0.2 * t)
                    w = jnp.exp(e - t) * lane_ok
                    if p == 0:
                        wbuf[j, :] = w
                    for hh_ in range(hps):
                        wh = jnp.take(
                            w, jnp.full((16,), p * hps + hh_, jnp.int32))
                        for g in range(C // 16):
                            col = hh_ * C + g * 16
                            rows[j, pl.ds(col, 16)] = (
                                rows[j, pl.ds(col, 16)] * wh)
                    return _
                lax.fori_loop(0, KCH, _edge, 0)

                pltpu.sync_copy(rows, acc_sh.at[sidx.at[0]], add=True)
                if p == 0:
                    pltpu.sync_copy(wbuf, den_sh.at[sidx.at[0]], add=True)
                return c
            return lax.fori_loop(0, SUB // KCH, _chunk, c)
        lax.fori_loop(0, TILE_E // SUB, _sub, 0)

        plsc.subcore_barrier()

        pltpu.sync_copy(acc_sh.at[pl.ds(base, ROWS_PT)],
                        acc_out.at[cid, p, pl.ds(base, ROWS_PT)])
        if p == 0:
            pltpu.sync_copy(den_sh.at[pl.ds(base, ROWS_PT)],
                            den_out.at[cid, pl.ds(base, ROWS_PT)])
        plsc.subcore_barrier()


def _sc_layer(nslab, slabw, hh):
    mesh = plsc.VectorSubcoreMesh(core_axis_name="c", subcore_axis_name="s",
                                  num_cores=NSC, num_subcores=NTILES)
    return pl.kernel(
        functools.partial(_sc_body, nslab, slabw, hh),
        out_type=[
            jax.ShapeDtypeStruct((NSC, nslab, NP, slabw), jnp.float32),
            jax.ShapeDtypeStruct((NSC, NP, 16), jnp.float32),
        ],
        mesh=mesh,
        compiler_params=pltpu.CompilerParams(use_tc_tiling_on_sc=False),
        scratch_types=[
            pltpu.VMEM((SUB,), jnp.int32),
            pltpu.VMEM((SUB,), jnp.int32),
            pltpu.VMEM((1, KCH), jnp.int32),
            pltpu.VMEM((1, KCH), jnp.int32),
            pltpu.VMEM((1, KCH), jnp.int32),
            pltpu.VMEM((KCH, slabw), jnp.float32),
            pltpu.VMEM((KCH, 16), jnp.float32),
            pltpu.VMEM((KCH, 16), jnp.float32),
            pltpu.VMEM((KCH, 16), jnp.float32),
            pltpu.VMEM((16,), jnp.float32),
            pltpu.VMEM_SHARED((NP, slabw), jnp.float32),
            pltpu.VMEM_SHARED((NP, 16), jnp.float32),
            pltpu.SemaphoreType.DMA,
        ],
    )


def _sc_run(sc_fn, o, d0, srcp, dstp, amax16):
    nslab = max(d0 // 128, 1)
    slabw = d0 // nslab
    asd = o[:, d0:d0 + 16]
    slabs = [o[:, i * slabw:(i + 1) * slabw] for i in range(nslab)]
    acc2, den2 = sc_fn(*slabs, asd, amax16, srcp, dstp)
    acc = acc2[0] + acc2[1]
    acc = jnp.moveaxis(acc, 0, 1).reshape(NP, d0)
    den = den2[0] + den2[1]
    return acc, den


def _augment(w, att_s, att_d, heads):
    f = w.shape[0]
    d0 = w.shape[1]
    c = d0 // heads
    wr = w.reshape(f, heads, c)
    ws = jnp.einsum("fhc,hc->fh", wr, att_s)
    wd = jnp.einsum("fhc,hc->fh", wr, att_d)
    pad = jnp.zeros((f, 8 - heads), jnp.float32)
    tail = jnp.zeros((f, 112), jnp.float32)
    return jnp.concatenate([w, ws, pad, wd, pad, tail], axis=1)


def kernel(x, edge_index, W1, as1, ad1, b1, W2, as2, ad2, b2,
           W3, as3, ad3, b3, linW, linb):
    x_p = jnp.zeros((NP, F_IN), jnp.float32).at[:N].set(x)
    src = edge_index[0].astype(jnp.int32)
    dst = edge_index[1].astype(jnp.int32)
    pad_e = EP - E
    srcp = jnp.concatenate([src, jnp.zeros((pad_e,), jnp.int32)])
    dstp = jnp.concatenate([dst, jnp.full((pad_e,), NP - 1, jnp.int32)])

    w1a = _augment(W1, as1, ad1, H)
    w2a = _augment(W2, as2, ad2, H)
    w3a = _augment(W3, as3, ad3, 1)

    sc_big = _sc_layer(4, 128, H)
    sc_small = _sc_layer(1, C, 1)

    o1, am1 = _mm(x_p, w1a, H * C)
    acc1, den1 = _sc_run(sc_big, o1, H * C, srcp, dstp, am1[0, :16])

    o2, am2 = _fin_mm(o1, acc1, den1, am1, b1.reshape(1, -1), w2a, H * C, H)
    acc2, den2 = _sc_run(sc_big, o2, H * C, srcp, dstp, am2[0, :16])

    o3, am3 = _fin_mm(o2, acc2, den2, am2, b2.reshape(1, -1), w3a, H * C, H)
    acc3, den3 = _sc_run(sc_small, o3, C, srcp, dstp, am3[0, :16])

    out = _head(o3, acc3, den3, am3, b3.reshape(1, -1), linW,
                linb.reshape(1, -1))
    return out[:N]


# speedup vs baseline: 14.4342x; 1.0400x over previous
import functools

import jax
import jax.numpy as jnp
from jax import lax
from jax.experimental import pallas as pl
from jax.experimental.pallas import tpu as pltpu
from jax.experimental.pallas import tpu_sc as plsc

N = 10000
E = 320000
F_IN = 128
H = 8
C = 64

NP = 10240
NSC = 2
NTILES = 16
EP = 327680
TILE_E = EP // NSC // NTILES
SUB = 2048
KCH = 64
BM = 512
ROWS_PT = NP // NTILES


def _lrelu(v):
    return jnp.where(v >= 0.0, v, 0.2 * v)


def _mm_body(d0, z_ref, w_ref, o_ref, am_ref):
    o = jnp.dot(z_ref[...], w_ref[...], preferred_element_type=jnp.float32)
    o_ref[...] = o
    bm = jnp.max(o[:, d0:d0 + 128], axis=0)
    bm8 = jnp.broadcast_to(bm[None, :], (8, 128))
    @pl.when(pl.program_id(0) == 0)
    def _():
        am_ref[...] = bm8
    @pl.when(pl.program_id(0) > 0)
    def _():
        am_ref[...] = jnp.maximum(am_ref[...], bm8)


def _mm(z, w_aug, d0):
    k = z.shape[1]
    dw = d0 + 128
    return pl.pallas_call(
        functools.partial(_mm_body, d0),
        grid=(NP // BM,),
        in_specs=[
            pl.BlockSpec((BM, k), lambda i: (i, 0)),
            pl.BlockSpec((k, dw), lambda i: (0, 0)),
        ],
        out_specs=[
            pl.BlockSpec((BM, dw), lambda i: (i, 0)),
            pl.BlockSpec((8, 128), lambda i: (0, 0)),
        ],
        out_shape=[
            jax.ShapeDtypeStruct((NP, dw), jnp.float32),
            jax.ShapeDtypeStruct((8, 128), jnp.float32),
        ],
    )(z, w_aug)


def _fin_mm_body(d0p, hp, d0n, op_ref, acc_ref, den_ref, am_ref, b_ref,
                 w_ref, o_ref, amo_ref):
    op = op_ref[...]
    h = op[:, :d0p]
    a_s = op[:, d0p:d0p + hp]
    a_d = op[:, d0p + 8:d0p + 8 + hp]
    a_glob = am_ref[0:1, 0:hp]
    w_self = jnp.exp(_lrelu(a_s + a_d) - _lrelu(a_glob + a_d))
    dent = den_ref[:, 0:hp] + w_self
    cph = d0p // hp
    sel = (lax.broadcasted_iota(jnp.int32, (hp, d0p), 1) // cph
           == lax.broadcasted_iota(jnp.int32, (hp, d0p), 0)
           ).astype(jnp.float32)
    wf = jnp.dot(w_self, sel, preferred_element_type=jnp.float32)
    df = jnp.dot(dent, sel, preferred_element_type=jnp.float32)
    z = (acc_ref[...] + wf * h) / df + b_ref[0:1, :]
    z = jnp.where(z > 0.0, z, jnp.exp(z) - 1.0)
    o = jnp.dot(z, w_ref[...], preferred_element_type=jnp.float32)
    o_ref[...] = o
    bm = jnp.max(o[:, d0n:d0n + 128], axis=0)
    bm8 = jnp.broadcast_to(bm[None, :], (8, 128))
    @pl.when(pl.program_id(0) == 0)
    def _():
        amo_ref[...] = bm8
    @pl.when(pl.program_id(0) > 0)
    def _():
        amo_ref[...] = jnp.maximum(amo_ref[...], bm8)


def _fin_mm(o_prev, acc, den, amax, bias2d, w_aug, d0p, hp):
    dwp = o_prev.shape[1]
    d0n = w_aug.shape[1] - 128
    return pl.pallas_call(
        functools.partial(_fin_mm_body, d0p, hp, d0n),
        grid=(NP // BM,),
        in_specs=[
            pl.BlockSpec((BM, dwp), lambda i: (i, 0)),
            pl.BlockSpec((BM, d0p), lambda i: (i, 0)),
            pl.BlockSpec((BM, 16), lambda i: (i, 0)),
            pl.BlockSpec((8, 128), lambda i: (0, 0)),
            pl.BlockSpec((1, d0p), lambda i: (0, 0)),
            pl.BlockSpec((w_aug.shape[0], w_aug.shape[1]), lambda i: (0, 0)),
        ],
        out_specs=[
            pl.BlockSpec((BM, w_aug.shape[1]), lambda i: (i, 0)),
            pl.BlockSpec((8, 128), lambda i: (0, 0)),
        ],
        out_shape=[
            jax.ShapeDtypeStruct((NP, w_aug.shape[1]), jnp.float32),
            jax.ShapeDtypeStruct((8, 128), jnp.float32),
        ],
    )(o_prev, acc, den, amax, bias2d, w_aug)


def _head_body(o3_ref, acc_ref, den_ref, am_ref, b_ref, lw_ref, lb_ref,
               out_ref):
    o3 = o3_ref[...]
    h = o3[:, :C]
    a_s = o3[:, C:C + 1]
    a_d = o3[:, C + 8:C + 9]
    a_glob = am_ref[0:1, 0:1]
    w_self = jnp.exp(_lrelu(a_s + a_d) - _lrelu(a_glob + a_d))
    dent = den_ref[:, 0:1] + w_self
    z = (acc_ref[...] + w_self * h) / dent + b_ref[0:1, :]
    lg = jnp.dot(z, lw_ref[...], preferred_element_type=jnp.float32)
    lg = lg + lb_ref[0:1, :]
    mx = jnp.max(lg, axis=1, keepdims=True)
    s = lg - mx
    out_ref[...] = s - jnp.log(jnp.sum(jnp.exp(s), axis=1, keepdims=True))


def _head(o3, acc3, den3, am3, b3_2d, lin_w, lin_b2d):
    dwp = o3.shape[1]
    return pl.pallas_call(
        _head_body,
        grid=(NP // BM,),
        in_specs=[
            pl.BlockSpec((BM, dwp), lambda i: (i, 0)),
            pl.BlockSpec((BM, C), lambda i: (i, 0)),
            pl.BlockSpec((BM, 16), lambda i: (i, 0)),
            pl.BlockSpec((8, 128), lambda i: (0, 0)),
            pl.BlockSpec((1, C), lambda i: (0, 0)),
            pl.BlockSpec((C, 2), lambda i: (0, 0)),
            pl.BlockSpec((1, 2), lambda i: (0, 0)),
        ],
        out_specs=pl.BlockSpec((BM, 2), lambda i: (i, 0)),
        out_shape=jax.ShapeDtypeStruct((NP, 2), jnp.float32),
    )(o3, acc3, den3, am3, b3_2d, lin_w, lin_b2d)


def _sc_body(nslab, slabw, hh, *refs):
    hslabs = list(refs[:nslab])
    (asd_hbm, amax_hbm, src_hbm, dst_hbm, acc_out, den_out,
     stage_s, stage_d, gidx_s, gidx_d, sidx,
     rows0, rows1, arow0, arow1, brow0, brow1, wbuf0, wbuf1,
     amv, acc_sh, den_sh, sem, sem1) = refs[nslab:]
    rows_b = [rows0, rows1]
    arow_b = [arow0, arow1]
    brow_b = [brow0, brow1]
    wbuf_b = [wbuf0, wbuf1]
    sem_b = [sem, sem1]
    hps = slabw // C
    cid = lax.axis_index("c")
    sid = lax.axis_index("s")
    pltpu.sync_copy(amax_hbm, amv)
    amv_val = amv[...]
    ebase = (cid * NTILES + sid) * TILE_E

    for p in range(nslab):
        rows = rows_b[0]
        wbuf = wbuf_b[0]
        def _z_rows(j, c):
            for g in range(slabw // 16):
                rows[j, pl.ds(g * 16, 16)] = jnp.zeros((16,), jnp.float32)
            if p == 0:
                wbuf[j, :] = jnp.zeros((16,), jnp.float32)
            return c
        lax.fori_loop(0, KCH, _z_rows, 0)
        base = sid * ROWS_PT
        for r in range(ROWS_PT // KCH):
            pltpu.sync_copy(rows.at[pl.ds(0, KCH)],
                            acc_sh.at[pl.ds(base + r * KCH, KCH)])
            if p == 0:
                pltpu.sync_copy(wbuf.at[pl.ds(0, KCH)],
                                den_sh.at[pl.ds(base + r * KCH, KCH)])
        plsc.subcore_barrier()

        def _sub(si, c):
            off = pl.multiple_of(ebase + si * SUB, 8)
            pltpu.sync_copy(src_hbm.at[pl.ds(off, SUB)], stage_s)
            pltpu.sync_copy(dst_hbm.at[pl.ds(off, SUB)], stage_d)

            def _pair(ci, c):
                gds = []
                for b in range(2):
                    c0 = (2 * ci + b) * KCH
                    for g in range(KCH // 16):
                        sv = stage_s[pl.ds(c0 + g * 16, 16)]
                        dv = stage_d[pl.ds(c0 + g * 16, 16)]
                        gidx_s[b, pl.ds(g * 16, 16)] = sv
                        gidx_d[b, pl.ds(g * 16, 16)] = dv
                        sidx[b, pl.ds(g * 16, 16)] = dv
                for b in range(2):
                    gds.append(pltpu.async_copy(
                        hslabs[p].at[gidx_s.at[b]], rows_b[b], sem_b[b]))
                    gds.append(pltpu.async_copy(
                        asd_hbm.at[gidx_s.at[b]], arow_b[b], sem_b[b]))
                    gds.append(pltpu.async_copy(
                        asd_hbm.at[gidx_d.at[b]], brow_b[b], sem_b[b]))

                sds = []
                for b in range(2):
                    rows = rows_b[b]
                    arow = arow_b[b]
                    brow = brow_b[b]
                    wbuf = wbuf_b[b]
                    for d in gds[3 * b:3 * b + 3]:
                        d.wait()

                    def _edge(j, _):
                        lane_i = lax.iota(jnp.int32, 16)
                        lane_ok = jnp.where(lane_i < hh, 1.0, 0.0)
                        av = arow[j, :]
                        bv = brow[j, :]
                        adv = jnp.take(bv, (lane_i + 8) % 16)
                        e = av + adv
                        e = jnp.where(e >= 0., e, 0.2 * e)
                        t = amv_val + adv
                        t = jnp.where(t >= 0., t, 0.2 * t)
                        w = jnp.exp(e - t) * lane_ok
                        if p == 0:
                            wbuf[j, :] = w
                        for hh_ in range(hps):
                            wh = jnp.take(
                                w, jnp.full((16,), p * hps + hh_, jnp.int32))
                            for g in range(C // 16):
                                col = hh_ * C + g * 16
                                rows[j, pl.ds(col, 16)] = (
                                    rows[j, pl.ds(col, 16)] * wh)
                        return _
                    lax.fori_loop(0, KCH, _edge, 0)

                    sds.append(pltpu.async_copy(
                        rows, acc_sh.at[sidx.at[b]], sem_b[b], add=True))
                    if p == 0:
                        sds.append(pltpu.async_copy(
                            wbuf, den_sh.at[sidx.at[b]], sem_b[b], add=True))
                for d in sds:
                    d.wait()
                return c
            return lax.fori_loop(0, SUB // KCH // 2, _pair, c)
        lax.fori_loop(0, TILE_E // SUB, _sub, 0)

        plsc.subcore_barrier()

        pltpu.sync_copy(acc_sh.at[pl.ds(base, ROWS_PT)],
                        acc_out.at[cid, p, pl.ds(base, ROWS_PT)])
        if p == 0:
            pltpu.sync_copy(den_sh.at[pl.ds(base, ROWS_PT)],
                            den_out.at[cid, pl.ds(base, ROWS_PT)])
        plsc.subcore_barrier()


def _sc_layer(nslab, slabw, hh):
    mesh = plsc.VectorSubcoreMesh(core_axis_name="c", subcore_axis_name="s",
                                  num_cores=NSC, num_subcores=NTILES)
    return pl.kernel(
        functools.partial(_sc_body, nslab, slabw, hh),
        out_type=[
            jax.ShapeDtypeStruct((NSC, nslab, NP, slabw), jnp.float32),
            jax.ShapeDtypeStruct((NSC, NP, 16), jnp.float32),
        ],
        mesh=mesh,
        compiler_params=pltpu.CompilerParams(use_tc_tiling_on_sc=False),
        scratch_types=[
            pltpu.VMEM((SUB,), jnp.int32),
            pltpu.VMEM((SUB,), jnp.int32),
            pltpu.VMEM((2, KCH), jnp.int32),
            pltpu.VMEM((2, KCH), jnp.int32),
            pltpu.VMEM((2, KCH), jnp.int32),
            pltpu.VMEM((KCH, slabw), jnp.float32),
            pltpu.VMEM((KCH, slabw), jnp.float32),
            pltpu.VMEM((KCH, 16), jnp.float32),
            pltpu.VMEM((KCH, 16), jnp.float32),
            pltpu.VMEM((KCH, 16), jnp.float32),
            pltpu.VMEM((KCH, 16), jnp.float32),
            pltpu.VMEM((KCH, 16), jnp.float32),
            pltpu.VMEM((KCH, 16), jnp.float32),
            pltpu.VMEM((16,), jnp.float32),
            pltpu.VMEM_SHARED((NP, slabw), jnp.float32),
            pltpu.VMEM_SHARED((NP, 16), jnp.float32),
            pltpu.SemaphoreType.DMA,
            pltpu.SemaphoreType.DMA,
        ],
    )


def _sc_run(sc_fn, o, d0, srcp, dstp, amax16):
    nslab = max(d0 // 128, 1)
    slabw = d0 // nslab
    asd = o[:, d0:d0 + 16]
    slabs = [o[:, i * slabw:(i + 1) * slabw] for i in range(nslab)]
    acc2, den2 = sc_fn(*slabs, asd, amax16, srcp, dstp)
    acc = acc2[0] + acc2[1]
    acc = jnp.moveaxis(acc, 0, 1).reshape(NP, d0)
    den = den2[0] + den2[1]
    return acc, den


def _augment(w, att_s, att_d, heads):
    f = w.shape[0]
    d0 = w.shape[1]
    c = d0 // heads
    wr = w.reshape(f, heads, c)
    ws = jnp.einsum("fhc,hc->fh", wr, att_s)
    wd = jnp.einsum("fhc,hc->fh", wr, att_d)
    pad = jnp.zeros((f, 8 - heads), jnp.float32)
    tail = jnp.zeros((f, 112), jnp.float32)
    return jnp.concatenate([w, ws, pad, wd, pad, tail], axis=1)


def kernel(x, edge_index, W1, as1, ad1, b1, W2, as2, ad2, b2,
           W3, as3, ad3, b3, linW, linb):
    x_p = jnp.zeros((NP, F_IN), jnp.float32).at[:N].set(x)
    src = edge_index[0].astype(jnp.int32)
    dst = edge_index[1].astype(jnp.int32)
    pad_e = EP - E
    srcp = jnp.concatenate([src, jnp.zeros((pad_e,), jnp.int32)])
    dstp = jnp.concatenate([dst, jnp.full((pad_e,), NP - 1, jnp.int32)])

    w1a = _augment(W1, as1, ad1, H)
    w2a = _augment(W2, as2, ad2, H)
    w3a = _augment(W3, as3, ad3, 1)

    sc_big = _sc_layer(4, 128, H)
    sc_small = _sc_layer(1, C, 1)

    o1, am1 = _mm(x_p, w1a, H * C)
    acc1, den1 = _sc_run(sc_big, o1, H * C, srcp, dstp, am1[0, :16])

    o2, am2 = _fin_mm(o1, acc1, den1, am1, b1.reshape(1, -1), w2a, H * C, H)
    acc2, den2 = _sc_run(sc_big, o2, H * C, srcp, dstp, am2[0, :16])

    o3, am3 = _fin_mm(o2, acc2, den2, am2, b2.reshape(1, -1), w3a, H * C, H)
    acc3, den3 = _sc_run(sc_small, o3, C, srcp, dstp, am3[0, :16])

    out = _head(o3, acc3, den3, am3, b3.reshape(1, -1), linW,
                linb.reshape(1, -1))
    return out[:N]
